# Initial kernel scaffold; baseline (speedup 1.0000x reference)
import jax
import jax.numpy as jnp
from jax.experimental import pallas as pl


def kernel(x, edge_index, W1, b1, W_mu, b_mu, W_ls, b_ls):
    raise NotImplementedError("write your pallas kernel here")


# speedup vs baseline: 15.4390x; 15.4390x over previous
import functools

import jax
import jax.numpy as jnp
from jax import lax
from jax.experimental import pallas as pl
from jax.experimental.pallas import tpu as pltpu
from jax.experimental.pallas import tpu_sc as plsc

N_NODES = 10000
NC = 2
NS = 16
NT = NC * NS
CHUNK = 128

ACC_PER_TILE = 640
ACC_ROWS = ACC_PER_TILE * NS
DUMMY = N_NODES

_MESH = plsc.VectorSubcoreMesh(core_axis_name="c", subcore_axis_name="s")


def _zero_rows(buf, nrows, width):
    def row(i, carry):
        def lane(j, carry2):
            buf[i, pl.ds(j * 16, 16)] = jnp.zeros((16,), jnp.float32)
            return carry2
        return lax.fori_loop(0, width // 16, lane, carry)
    lax.fori_loop(0, nrows, row, 0)


def _deg_body(dst_hbm, out_hbm, dst_v, ones_v, acc):
    c = lax.axis_index("c")
    s = lax.axis_index("s")
    wid = c * NS + s
    nchunks = dst_hbm.shape[1]

    _zero_rows(ones_v, CHUNK, 16)
    for k in range(ACC_PER_TILE // CHUNK):
        pltpu.sync_copy(ones_v.at[pl.ds(0, CHUNK)],
                        acc.at[pl.ds(s * ACC_PER_TILE + k * CHUNK, CHUNK)])
    def row(i, carry):
        ones_v[i, pl.ds(0, 16)] = jnp.ones((16,), jnp.float32)
        return carry
    lax.fori_loop(0, CHUNK, row, 0)
    plsc.subcore_barrier()

    pltpu.sync_copy(dst_hbm.at[wid], dst_v)
    def chunk(j, carry):
        pltpu.sync_copy(ones_v, acc.at[dst_v.at[j]], add=True)
        return carry
    lax.fori_loop(0, nchunks, chunk, 0)
    plsc.subcore_barrier()

    pltpu.sync_copy(acc.at[pl.ds(s * ACC_PER_TILE, ACC_PER_TILE)],
                    out_hbm.at[c, pl.ds(s * ACC_PER_TILE, ACC_PER_TILE)])


def _make_deg(nchunks):
    return pl.kernel(
        _deg_body,
        out_type=jax.ShapeDtypeStruct((NC, ACC_ROWS, 16), jnp.float32),
        mesh=_MESH,
        scratch_types=[
            pltpu.VMEM((nchunks, CHUNK), jnp.int32),
            pltpu.VMEM((CHUNK, 16), jnp.float32),
            pltpu.VMEM_SHARED((ACC_ROWS, 16), jnp.float32),
        ],
    )


def _agg_body(table, src_hbm, dst_hbm, out_hbm, src_v, dst_v, rows_v, sem, acc):
    c = lax.axis_index("c")
    s = lax.axis_index("s")
    wid = c * NS + s
    nchunks = src_hbm.shape[1]

    _zero_rows(rows_v, CHUNK, 128)
    for k in range(ACC_PER_TILE // CHUNK):
        pltpu.sync_copy(rows_v,
                        acc.at[pl.ds(s * ACC_PER_TILE + k * CHUNK, CHUNK)])
    plsc.subcore_barrier()

    pltpu.sync_copy(src_hbm.at[wid], src_v)
    pltpu.sync_copy(dst_hbm.at[wid], dst_v)
    def chunk(j, carry):
        pltpu.async_copy(table.at[src_v.at[j]], rows_v, sem).wait()
        pltpu.sync_copy(rows_v, acc.at[dst_v.at[j]], add=True)
        return carry
    lax.fori_loop(0, nchunks, chunk, 0)
    plsc.subcore_barrier()

    pltpu.sync_copy(acc.at[pl.ds(s * ACC_PER_TILE, ACC_PER_TILE)],
                    out_hbm.at[c, pl.ds(s * ACC_PER_TILE, ACC_PER_TILE)])


def _make_agg(nchunks):
    return pl.kernel(
        _agg_body,
        out_type=jax.ShapeDtypeStruct((NC, ACC_ROWS, 128), jnp.float32),
        mesh=_MESH,
        scratch_types=[
            pltpu.VMEM((nchunks, CHUNK), jnp.int32),
            pltpu.VMEM((nchunks, CHUNK), jnp.int32),
            pltpu.VMEM((CHUNK, 128), jnp.float32),
            pltpu.SemaphoreType.DMA,
            pltpu.VMEM_SHARED((ACC_ROWS, 128), jnp.float32),
        ],
    )


BLK = 1000


def _tc1_body(x_ref, w_ref, deg_ref, xs_ref):
    d = deg_ref[0] + deg_ref[1]
    dinv = lax.rsqrt(d[:, 0:1] + 1.0)
    xw = jnp.dot(x_ref[...], w_ref[...], preferred_element_type=jnp.float32)
    xs_ref[...] = xw * dinv


def _tc2_body(z_ref, xs_ref, deg_ref, b_ref, out_ref):
    d = deg_ref[0] + deg_ref[1]
    dinv = lax.rsqrt(d[:, 0:1] + 1.0)
    y = (z_ref[0] + z_ref[1] + xs_ref[...]) * dinv + b_ref[...]
    out_ref[...] = jnp.maximum(y, 0.0) * dinv


def _tc3_body(z_ref, hs_ref, deg_ref, w_ref, b_ref, out_ref):
    d = deg_ref[0] + deg_ref[1]
    dinv = lax.rsqrt(d[:, 0:1] + 1.0)
    agg = (z_ref[0] + z_ref[1] + hs_ref[...]) * dinv
    out_ref[...] = (
        jnp.dot(agg, w_ref[...], preferred_element_type=jnp.float32)
        + b_ref[...]
    )


def _node_spec(width):
    return pl.BlockSpec((BLK, width), lambda i: (i, 0))


def _pair_spec(width):
    return pl.BlockSpec((NC, BLK, width), lambda i: (0, i, 0))


def _full_spec(r, c):
    return pl.BlockSpec((r, c), lambda i: (0, 0))


_GRID = (N_NODES // BLK,)


def _tc1(x, w1, deg):
    return pl.pallas_call(
        _tc1_body,
        grid=_GRID,
        in_specs=[_node_spec(128), _full_spec(128, 128), _pair_spec(16)],
        out_specs=_node_spec(128),
        out_shape=jax.ShapeDtypeStruct((N_NODES, 128), jnp.float32),
    )(x, w1, deg)


def _tc2(z, xs, deg, b1):
    return pl.pallas_call(
        _tc2_body,
        grid=_GRID,
        in_specs=[_pair_spec(128), _node_spec(128), _pair_spec(16),
                  _full_spec(1, 128)],
        out_specs=_node_spec(128),
        out_shape=jax.ShapeDtypeStruct((N_NODES, 128), jnp.float32),
    )(z, xs, deg, b1)


def _tc3(z, hs, deg, wcat, bcat):
    return pl.pallas_call(
        _tc3_body,
        grid=_GRID,
        in_specs=[_pair_spec(128), _node_spec(128), _pair_spec(16),
                  _full_spec(128, 128), _full_spec(1, 128)],
        out_specs=_node_spec(128),
        out_shape=jax.ShapeDtypeStruct((N_NODES, 128), jnp.float32),
    )(z, hs, deg, wcat, bcat)


def kernel(x, edge_index, W1, b1, W_mu, b_mu, W_ls, b_ls):
    num_edges = edge_index.shape[1]
    per_tile = -(-num_edges // (NT * CHUNK)) * CHUNK
    nchunks = per_tile // CHUNK
    pad = NT * per_tile - num_edges

    src = jnp.concatenate(
        [edge_index[0], jnp.zeros((pad,), jnp.int32)]).reshape(NT, nchunks, CHUNK)
    dst = jnp.concatenate(
        [edge_index[1], jnp.full((pad,), DUMMY, jnp.int32)]).reshape(NT, nchunks, CHUNK)

    deg = _make_deg(nchunks)(dst)

    xs1 = _tc1(x, W1, deg)
    agg = _make_agg(nchunks)
    z1 = agg(xs1, src, dst)
    h1s = _tc2(z1, xs1, deg, b1.reshape(1, 128))
    z2 = agg(h1s, src, dst)

    wcat = jnp.concatenate([W_mu, W_ls], axis=1)
    bcat = jnp.concatenate([b_mu, b_ls]).reshape(1, 128)
    out = _tc3(z2, h1s, deg, wcat, bcat)
    return out[:, :64], out[:, 64:]


# speedup vs baseline: 36.7193x; 2.3784x over previous
import jax
import jax.numpy as jnp
from jax import lax
from jax.experimental import pallas as pl
from jax.experimental.pallas import tpu as pltpu
from jax.experimental.pallas import tpu_sc as plsc

N_NODES = 10000
NC = 2
NS = 16
NT = NC * NS
CHUNK = 128
NBUF = 2
SB = 16

ACC_PER_TILE = 640
N_PAD = ACC_PER_TILE * NS

_MESH = plsc.VectorSubcoreMesh(core_axis_name="c", subcore_axis_name="s")


def _zero_rows(buf, nrows, width):
    def row(i, carry):
        def lane(j, carry2):
            buf[i, pl.ds(j * 16, 16)] = jnp.zeros((16,), jnp.float32)
            return carry2
        return lax.fori_loop(0, width // 16, lane, carry)
    lax.fori_loop(0, nrows, row, 0)


def _deg_body(dst_hbm, out_hbm, dst_v, ones_v, acc):
    c = lax.axis_index("c")
    s = lax.axis_index("s")
    wid = c * NS + s
    nreal = dst_hbm.shape[1] - SB

    _zero_rows(ones_v, CHUNK, 16)
    for k in range(ACC_PER_TILE // CHUNK):
        pltpu.sync_copy(ones_v.at[pl.ds(0, CHUNK)],
                        acc.at[pl.ds(s * ACC_PER_TILE + k * CHUNK, CHUNK)])
    def row(i, carry):
        ones_v[i, pl.ds(0, 16)] = jnp.ones((16,), jnp.float32)
        return carry
    lax.fori_loop(0, CHUNK, row, 0)
    plsc.subcore_barrier()

    pltpu.sync_copy(dst_hbm.at[wid], dst_v)
    def chunk(j, carry):
        pltpu.sync_copy(ones_v, acc.at[dst_v.at[j]], add=True)
        return carry
    lax.fori_loop(0, nreal, chunk, 0)
    plsc.subcore_barrier()

    pltpu.sync_copy(acc.at[pl.ds(s * ACC_PER_TILE, ACC_PER_TILE)],
                    out_hbm.at[c, pl.ds(s * ACC_PER_TILE, ACC_PER_TILE)])


def _make_deg(nchunks):
    return pl.kernel(
        _deg_body,
        out_type=jax.ShapeDtypeStruct((NC, N_PAD, 16), jnp.float32),
        mesh=_MESH,
        scratch_types=[
            pltpu.VMEM((nchunks, CHUNK), jnp.int32),
            pltpu.VMEM((CHUNK, 16), jnp.float32),
            pltpu.VMEM_SHARED((N_PAD, 16), jnp.float32),
        ],
    )


def _agg_body(table, src_hbm, dst_hbm, out_hbm, src_v, dst_v, rows_v,
              gsem, isem, acc):
    c = lax.axis_index("c")
    s = lax.axis_index("s")
    wid = c * NS + s
    nsb = (src_hbm.shape[1] - SB) // SB

    _zero_rows(rows_v, CHUNK, 128)
    for k in range(ACC_PER_TILE // CHUNK):
        pltpu.sync_copy(rows_v.at[pl.ds(0, CHUNK)],
                        acc.at[pl.ds(s * ACC_PER_TILE + k * CHUNK, CHUNK)])
    plsc.subcore_barrier()

    def _buf(b):
        return rows_v.at[pl.ds(b * CHUNK, CHUNK)]

    def _idx_start(sb, p):
        pltpu.async_copy(src_hbm.at[wid, pl.ds(sb * SB, SB)],
                         src_v.at[p], isem)
        pltpu.async_copy(dst_hbm.at[wid, pl.ds(sb * SB, SB)],
                         dst_v.at[p], isem)

    def _idx_wait():
        pltpu.make_async_copy(src_hbm.at[0, pl.ds(0, SB)],
                              src_v.at[0], isem).wait()
        pltpu.make_async_copy(dst_hbm.at[0, pl.ds(0, SB)],
                              dst_v.at[0], isem).wait()

    def _gather_start(p, k, b):
        pltpu.async_copy(table.at[src_v.at[p, k]], _buf(b), gsem)

    def _gather_wait(b):
        pltpu.make_async_copy(table.at[pl.ds(0, CHUNK)], _buf(b), gsem).wait()

    _idx_start(0, 0)
    def superblock(sb, carry):
        p = lax.rem(sb, 2)
        _idx_wait()
        _idx_start(sb + 1, 1 - p)
        for b in range(NBUF):
            _gather_start(p, b, b)
        for j in range(SB):
            b = j % NBUF
            _gather_wait(b)
            pltpu.sync_copy(_buf(b), acc.at[dst_v.at[p, j]], add=True)
            if j + NBUF < SB:
                _gather_start(p, j + NBUF, b)
        return carry
    lax.fori_loop(0, nsb, superblock, 0)
    _idx_wait()
    plsc.subcore_barrier()

    pltpu.sync_copy(acc.at[pl.ds(s * ACC_PER_TILE, ACC_PER_TILE)],
                    out_hbm.at[c, pl.ds(s * ACC_PER_TILE, ACC_PER_TILE)])


def _make_agg(nchunks):
    return pl.kernel(
        _agg_body,
        out_type=jax.ShapeDtypeStruct((NC, N_PAD, 128), jnp.float32),
        mesh=_MESH,
        scratch_types=[
            pltpu.VMEM((2, SB, CHUNK), jnp.int32),
            pltpu.VMEM((2, SB, CHUNK), jnp.int32),
            pltpu.VMEM((NBUF * CHUNK, 128), jnp.float32),
            pltpu.SemaphoreType.DMA,
            pltpu.SemaphoreType.DMA,
            pltpu.VMEM_SHARED((N_PAD, 128), jnp.float32),
        ],
    )


BLK = 1024


def _rows(pid):
    return (lax.broadcasted_iota(jnp.int32, (BLK, 1), 0) + pid * BLK)


def _dinv(deg_ref, pad_edges):
    d = deg_ref[0][:, 0:1] + deg_ref[1][:, 0:1]
    ind = (_rows(pl.program_id(0)) < pad_edges).astype(jnp.float32)
    return lax.rsqrt(d - ind + 1.0)


def _tc1_body(pad_edges, x_ref, w_ref, deg_ref, xs_ref):
    dinv = _dinv(deg_ref, pad_edges)
    xw = jnp.dot(x_ref[...], w_ref[...], preferred_element_type=jnp.float32)
    xs_ref[...] = xw * dinv


def _tc2_body(pad_edges, z_ref, xs_ref, deg_ref, b_ref, out_ref):
    dinv = _dinv(deg_ref, pad_edges)
    y = (z_ref[0] + z_ref[1] + xs_ref[...]) * dinv + b_ref[...]
    valid = (_rows(pl.program_id(0)) < N_NODES).astype(jnp.float32)
    out_ref[...] = jnp.maximum(y, 0.0) * dinv * valid


def _tc3_body(pad_edges, z_ref, hs_ref, deg_ref, w_ref, b_ref, out_ref):
    dinv = _dinv(deg_ref, pad_edges)
    agg = (z_ref[0] + z_ref[1] + hs_ref[...]) * dinv
    out_ref[...] = (
        jnp.dot(agg, w_ref[...], preferred_element_type=jnp.float32)
        + b_ref[...]
    )


def _node_spec(width):
    return pl.BlockSpec((BLK, width), lambda i: (i, 0))


def _pair_spec(width):
    return pl.BlockSpec((NC, BLK, width), lambda i: (0, i, 0))


def _full_spec(r, c):
    return pl.BlockSpec((r, c), lambda i: (0, 0))


_GRID = (N_PAD // BLK,)


def _tc1(x, w1, deg, pad_edges):
    return pl.pallas_call(
        lambda *refs: _tc1_body(pad_edges, *refs),
        grid=_GRID,
        in_specs=[_node_spec(128), _full_spec(128, 128), _pair_spec(16)],
        out_specs=_node_spec(128),
        out_shape=jax.ShapeDtypeStruct((N_PAD, 128), jnp.float32),
    )(x, w1, deg)


def _tc2(z, xs, deg, b1, pad_edges):
    return pl.pallas_call(
        lambda *refs: _tc2_body(pad_edges, *refs),
        grid=_GRID,
        in_specs=[_pair_spec(128), _node_spec(128), _pair_spec(16),
                  _full_spec(1, 128)],
        out_specs=_node_spec(128),
        out_shape=jax.ShapeDtypeStruct((N_PAD, 128), jnp.float32),
    )(z, xs, deg, b1)


def _tc3(z, hs, deg, wcat, bcat, pad_edges):
    return pl.pallas_call(
        lambda *refs: _tc3_body(pad_edges, *refs),
        grid=_GRID,
        in_specs=[_pair_spec(128), _node_spec(128), _pair_spec(16),
                  _full_spec(128, 128), _full_spec(1, 128)],
        out_specs=_node_spec(128),
        out_shape=jax.ShapeDtypeStruct((N_PAD, 128), jnp.float32),
    )(z, hs, deg, wcat, bcat)


def kernel(x, edge_index, W1, b1, W_mu, b_mu, W_ls, b_ls):
    num_edges = edge_index.shape[1]
    per_tile = num_edges // NT
    nreal = -(-(per_tile) // CHUNK)
    nreal = -(-nreal // SB) * SB
    nchunks = nreal + SB
    tile_pad = nreal * CHUNK - per_tile
    pad_edges = tile_pad * NT

    pad_src = N_PAD - 1 - (jnp.arange(pad_edges, dtype=jnp.int32)
                           % (N_PAD - N_NODES)).reshape(NT, tile_pad)
    pad_dst = jnp.arange(pad_edges, dtype=jnp.int32).reshape(NT, tile_pad)
    src = jnp.concatenate(
        [edge_index[0].reshape(NT, per_tile), pad_src], axis=1)
    dst = jnp.concatenate(
        [edge_index[1].reshape(NT, per_tile), pad_dst], axis=1)
    dummy = jnp.zeros((NT, SB * CHUNK), jnp.int32)
    src = jnp.concatenate([src, dummy], axis=1).reshape(NT, nchunks, CHUNK)
    dst = jnp.concatenate([dst, dummy], axis=1).reshape(NT, nchunks, CHUNK)

    xp = jnp.concatenate(
        [x, jnp.zeros((N_PAD - N_NODES, x.shape[1]), x.dtype)])

    deg = _make_deg(nchunks)(dst)
    xs1 = _tc1(xp, W1, deg, pad_edges)
    agg = _make_agg(nchunks)
    z1 = agg(xs1, src, dst)
    h1s = _tc2(z1, xs1, deg, b1.reshape(1, 128), pad_edges)
    z2 = agg(h1s, src, dst)

    wcat = jnp.concatenate([W_mu, W_ls], axis=1)
    bcat = jnp.concatenate([b_mu, b_ls]).reshape(1, 128)
    out = _tc3(z2, h1s, deg, wcat, bcat, pad_edges)
    return out[:N_NODES, :64], out[:N_NODES, 64:]
